# baseline (speedup 1.0000x reference)
import functools

import jax
import jax.numpy as jnp
import numpy as np
from jax import lax
from jax.experimental import pallas as pl
from jax.experimental.pallas import tpu as pltpu
from jax.experimental.pallas import tpu_sc as plsc

N_NODES = 10000
D_FEAT = 128
HIDDEN = 32
N_EDGES = 320000

NPAD = 10240
NS = 16
NHALF = 2
CHUNK = 128
NCH = 80
NE_TILE = NCH * CHUNK
EPAD = NHALF * NS * NE_TILE
ROWS_T = NPAD // NS
FIRE = 8

_f32 = jnp.float32


def _sc_mesh():
    return plsc.VectorSubcoreMesh(core_axis_name="c", subcore_axis_name="s",
                                  num_cores=1)


_SC_PARAMS = pltpu.CompilerParams(use_tc_tiling_on_sc=False,
                                  needs_layout_passes=False)


def _sc_scatter_rows(v, src3, dst3):
    @functools.partial(
        pl.kernel,
        out_type=jax.ShapeDtypeStruct((NPAD, HIDDEN), _f32),
        mesh=_sc_mesh(),
        compiler_params=_SC_PARAMS,
        scratch_types=[
            pltpu.VMEM((NCH, CHUNK), jnp.int32),
            pltpu.VMEM((NCH, CHUNK), jnp.int32),
            pltpu.VMEM((2, CHUNK, HIDDEN), _f32),
            pltpu.VMEM_SHARED((NPAD, HIDDEN), _f32),
            pltpu.SemaphoreType.DMA,
        ],
    )
    def k(v_hbm, src_hbm, dst_hbm, out_hbm, src_v, dst_v, bufs, acc_sh, gsem):
        sid = lax.axis_index("s")
        r0 = sid * ROWS_T
        pltpu.sync_copy(src_hbm.at[sid], src_v)
        pltpu.sync_copy(dst_hbm.at[sid], dst_v)
        pltpu.sync_copy(v_hbm.at[pl.ds(r0, ROWS_T)], acc_sh.at[pl.ds(r0, ROWS_T)])
        plsc.subcore_barrier()

        pltpu.make_async_copy(v_hbm.at[src_v.at[0]], bufs.at[0], gsem).start()

        def body(j, carry):
            slot = lax.rem(j, 2)
            nxt = lax.rem(j + 1, 2)

            @pl.when(j < NCH - 1)
            def _():
                pltpu.make_async_copy(
                    v_hbm.at[src_v.at[j + 1]], bufs.at[nxt], gsem).start()

            pltpu.make_async_copy(
                v_hbm.at[src_v.at[j]], bufs.at[slot], gsem).wait()
            pltpu.sync_copy(bufs.at[slot], acc_sh.at[dst_v.at[j]], add=True)
            return carry

        lax.fori_loop(0, NCH, body, 0, unroll=False)
        plsc.subcore_barrier()
        pltpu.sync_copy(acc_sh.at[pl.ds(r0, ROWS_T)],
                        out_hbm.at[pl.ds(r0, ROWS_T)])

    return k(v, src3, dst3)


def _sc_scatter_scalar(v, srcf, dst3, do_gather):
    @functools.partial(
        pl.kernel,
        out_type=jax.ShapeDtypeStruct((NPAD,), _f32),
        mesh=_sc_mesh(),
        compiler_params=_SC_PARAMS,
        scratch_types=[
            pltpu.VMEM((NE_TILE,), jnp.int32),
            pltpu.VMEM((NCH, CHUNK), jnp.int32),
            pltpu.VMEM((NPAD,), _f32),
            pltpu.VMEM((NE_TILE,), _f32),
            pltpu.VMEM_SHARED((NPAD,), _f32),
            pltpu.SemaphoreType.DMA,
        ],
    )
    def k(v_hbm, src_hbm, dst_hbm, out_hbm, src_v, dst_v, g_v, val_v, acc_sh, ssem):
        sid = lax.axis_index("s")
        r0 = sid * ROWS_T
        pltpu.sync_copy(dst_hbm.at[sid], dst_v)
        pltpu.sync_copy(v_hbm.at[pl.ds(r0, ROWS_T)], acc_sh.at[pl.ds(r0, ROWS_T)])
        if do_gather:
            pltpu.sync_copy(src_hbm.at[sid], src_v)
            pltpu.sync_copy(v_hbm, g_v)

            def gbody(i, carry):
                idx = src_v[pl.ds(i * 16, 16)]
                val_v[pl.ds(i * 16, 16)] = plsc.load_gather(g_v, [idx])
                return carry

            lax.fori_loop(0, NE_TILE // 16, gbody, 0, unroll=4)
        else:
            ones = jnp.full((16,), 1.0, dtype=_f32)
            for i in range(CHUNK // 16):
                val_v[pl.ds(i * 16, 16)] = ones
        plsc.subcore_barrier()

        def group(g, carry):
            base = g * FIRE
            for t in range(FIRE):
                off = (base + t) * CHUNK if do_gather else 0
                pltpu.make_async_copy(
                    val_v.at[pl.ds(off, CHUNK)],
                    acc_sh.at[dst_v.at[base + t]], ssem).start(add=True)
            for t in range(FIRE):
                off = (base + t) * CHUNK if do_gather else 0
                pltpu.make_async_copy(
                    val_v.at[pl.ds(off, CHUNK)],
                    acc_sh.at[dst_v.at[base + t]], ssem).wait()
            return carry

        lax.fori_loop(0, NCH // FIRE, group, 0, unroll=False)
        plsc.subcore_barrier()
        pltpu.sync_copy(acc_sh.at[pl.ds(r0, ROWS_T)],
                        out_hbm.at[pl.ds(r0, ROWS_T)])

    return k(v, srcf, dst3)


ROWS_B = 1280


def _tc_call(body, nout, *args):
    specs = []
    for a in args:
        if a.shape[0] == NPAD:
            specs.append(pl.BlockSpec((ROWS_B,) + a.shape[1:],
                                      lambda i: (i,) + (0,) * (a.ndim - 1)))
        else:
            specs.append(pl.BlockSpec(a.shape, lambda i: (0,) * a.ndim))
    out_shapes = [jax.ShapeDtypeStruct((NPAD, w), _f32) for w in nout]
    out_specs = [pl.BlockSpec((ROWS_B, w), lambda i: (i, 0)) for w in nout]
    res = pl.pallas_call(
        body,
        grid=(NPAD // ROWS_B,),
        in_specs=specs,
        out_specs=out_specs,
        out_shape=out_shapes,
    )(*args)
    return res


def _tc_deg_g1(x, W1, dp0, dp1):
    def body(x_ref, w_ref, d0_ref, d1_ref, dinv_ref, g1_ref):
        deg = d0_ref[...] + d1_ref[...] - 1.0
        dinv = lax.rsqrt(jnp.maximum(deg, 1.0))
        h0 = jnp.dot(x_ref[...], w_ref[...], preferred_element_type=_f32)
        dinv_ref[...] = dinv
        g1_ref[...] = dinv * h0

    return _tc_call(body, (1, HIDDEN), x, W1, dp0, dp1)


def _tc_h_g2(ap0, ap1, g1, dinv, b1, W2):
    def body(a0_ref, a1_ref, g1_ref, di_ref, b1_ref, w2_ref, g2_ref):
        dinv = di_ref[...]
        s = a0_ref[...] + a1_ref[...] - g1_ref[...]
        h = jnp.maximum(dinv * s + b1_ref[...], 0.0)
        z = jnp.dot(h, w2_ref[...], preferred_element_type=_f32)
        g2_ref[...] = dinv * z

    (g2,) = _tc_call(body, (1,), ap0, ap1, g1, dinv, b1, W2)
    return g2


def _tc_final(sp0, sp1, g2, dinv, b2):
    def body(s0_ref, s1_ref, g2_ref, di_ref, b2_ref, o_ref):
        s = s0_ref[...] + s1_ref[...] - g2_ref[...]
        o_ref[...] = di_ref[...] * s + b2_ref[...]

    (out,) = _tc_call(body, (1,), sp0, sp1, g2, dinv, b2)
    return out


_PAD_IDX = np.asarray(
    N_NODES + np.arange(EPAD - N_EDGES) % (NPAD - N_NODES), dtype=np.int32)

EHALF = EPAD // NHALF


def kernel(x, edge_index, W1, b1, W2, b2):
    ei = edge_index.astype(jnp.int32)
    pad_idx = jnp.asarray(_PAD_IDX)
    src = jnp.concatenate([ei[0], pad_idx])
    dst = jnp.concatenate([ei[1], pad_idx])
    src3 = [src[h * EHALF:(h + 1) * EHALF].reshape(NS, NCH, CHUNK)
            for h in range(NHALF)]
    dst3 = [dst[h * EHALF:(h + 1) * EHALF].reshape(NS, NCH, CHUNK)
            for h in range(NHALF)]
    srcf = [s.reshape(NS, NE_TILE) for s in src3]
    xp = jnp.pad(x, ((0, NPAD - N_NODES), (0, 0)))
    ones = jnp.ones((NPAD,), _f32)
    b1r = b1.reshape(1, HIDDEN)
    b2r = b2.reshape(1, 1)

    dp0 = _sc_scatter_scalar(ones, srcf[0], dst3[0], do_gather=False)
    dp1 = _sc_scatter_scalar(ones, srcf[1], dst3[1], do_gather=False)
    dinv, g1 = _tc_deg_g1(xp, W1, dp0.reshape(NPAD, 1), dp1.reshape(NPAD, 1))
    ap0 = _sc_scatter_rows(g1, src3[0], dst3[0])
    ap1 = _sc_scatter_rows(g1, src3[1], dst3[1])
    g2 = _tc_h_g2(ap0, ap1, g1, dinv, b1r, W2)
    sp0 = _sc_scatter_scalar(g2.reshape(NPAD), srcf[0], dst3[0], do_gather=True)
    sp1 = _sc_scatter_scalar(g2.reshape(NPAD), srcf[1], dst3[1], do_gather=True)
    out = _tc_final(sp0.reshape(NPAD, 1), sp1.reshape(NPAD, 1),
                    g2, dinv, b2r)
    return out[:N_NODES, 0]


# speedup vs baseline: 1.4969x; 1.4969x over previous
import functools

import jax
import jax.numpy as jnp
import numpy as np
from jax import lax
from jax.experimental import pallas as pl
from jax.experimental.pallas import tpu as pltpu
from jax.experimental.pallas import tpu_sc as plsc

N_NODES = 10000
D_FEAT = 128
HIDDEN = 32
N_EDGES = 320000

NPAD = 10240
NS = 16
CHUNK = 128
NCH = 160
NE_TILE = NCH * CHUNK
EPAD = NS * NE_TILE
ROWS_T = NPAD // NS
FIRE = 8
NBUF = 4

_f32 = jnp.float32


def _sc_mesh():
    return plsc.VectorSubcoreMesh(core_axis_name="c", subcore_axis_name="s",
                                  num_cores=1)


_SC_PARAMS = pltpu.CompilerParams(use_tc_tiling_on_sc=False,
                                  needs_layout_passes=False)


def _sc_deg(ones_hbm_shaped, dst3):
    @functools.partial(
        pl.kernel,
        out_type=jax.ShapeDtypeStruct((NPAD,), _f32),
        mesh=_sc_mesh(),
        compiler_params=_SC_PARAMS,
        scratch_types=[
            pltpu.VMEM((NCH, CHUNK), jnp.int32),
            pltpu.VMEM((CHUNK,), _f32),
            pltpu.VMEM_SHARED((NPAD,), _f32),
            pltpu.SemaphoreType.DMA,
        ],
    )
    def k(v_hbm, dst_hbm, out_hbm, dst_v, ones_v, acc_sh, ssem):
        sid = lax.axis_index("s")
        r0 = sid * ROWS_T
        pltpu.sync_copy(dst_hbm.at[sid], dst_v)
        pltpu.sync_copy(v_hbm.at[pl.ds(r0, ROWS_T)], acc_sh.at[pl.ds(r0, ROWS_T)])
        one16 = jnp.full((16,), 1.0, dtype=_f32)
        for i in range(CHUNK // 16):
            ones_v[pl.ds(i * 16, 16)] = one16
        plsc.subcore_barrier()

        def group(g, carry):
            base = g * FIRE
            for t in range(FIRE):
                pltpu.make_async_copy(
                    ones_v, acc_sh.at[dst_v.at[base + t]], ssem).start(add=True)
            for t in range(FIRE):
                pltpu.make_async_copy(
                    ones_v, acc_sh.at[dst_v.at[base + t]], ssem).wait()
            return carry

        lax.fori_loop(0, NCH // FIRE, group, 0, unroll=False)
        plsc.subcore_barrier()
        pltpu.sync_copy(acc_sh.at[pl.ds(r0, ROWS_T)], out_hbm.at[pl.ds(r0, ROWS_T)])

    return k(ones_hbm_shaped, dst3)


def _sc_scatter_rows(v, src3, dst3):
    @functools.partial(
        pl.kernel,
        out_type=jax.ShapeDtypeStruct((NPAD, HIDDEN), _f32),
        mesh=_sc_mesh(),
        compiler_params=_SC_PARAMS,
        scratch_types=[
            pltpu.VMEM((NCH, CHUNK), jnp.int32),
            pltpu.VMEM((NCH, CHUNK), jnp.int32),
            pltpu.VMEM((NBUF, CHUNK, HIDDEN), _f32),
            pltpu.VMEM_SHARED((NPAD, HIDDEN), _f32),
            pltpu.SemaphoreType.DMA,
            pltpu.SemaphoreType.DMA,
        ],
    )
    def k(v_hbm, src_hbm, dst_hbm, out_hbm, src_v, dst_v, bufs, acc_sh,
          gsem, ssem):
        sid = lax.axis_index("s")
        r0 = sid * ROWS_T
        pltpu.sync_copy(src_hbm.at[sid], src_v)
        pltpu.sync_copy(dst_hbm.at[sid], dst_v)
        pltpu.sync_copy(v_hbm.at[pl.ds(r0, ROWS_T)], acc_sh.at[pl.ds(r0, ROWS_T)])
        plsc.subcore_barrier()

        pltpu.make_async_copy(v_hbm.at[src_v.at[0]], bufs.at[0], gsem).start()
        pltpu.make_async_copy(v_hbm.at[src_v.at[1]], bufs.at[1], gsem).start()

        def body(j, carry):
            slot = lax.rem(j, NBUF)

            @pl.when(j >= 2)
            def _():
                pltpu.make_async_copy(
                    bufs.at[lax.rem(j - 2, NBUF)],
                    acc_sh.at[dst_v.at[j - 2]], ssem).wait()

            @pl.when(j < NCH - 2)
            def _():
                pltpu.make_async_copy(
                    v_hbm.at[src_v.at[j + 2]],
                    bufs.at[lax.rem(j + 2, NBUF)], gsem).start()

            pltpu.make_async_copy(
                v_hbm.at[src_v.at[j]], bufs.at[slot], gsem).wait()
            pltpu.make_async_copy(
                bufs.at[slot], acc_sh.at[dst_v.at[j]], ssem).start(add=True)
            return carry

        lax.fori_loop(0, NCH, body, 0, unroll=False)
        for j in (NCH - 2, NCH - 1):
            pltpu.make_async_copy(
                bufs.at[j % NBUF], acc_sh.at[dst_v.at[j]], ssem).wait()
        plsc.subcore_barrier()
        pltpu.sync_copy(acc_sh.at[pl.ds(r0, ROWS_T)],
                        out_hbm.at[pl.ds(r0, ROWS_T)])

    return k(v, src3, dst3)


def _sc_scatter_scalar_final(g2, dinv, b2b, srcf, dst3):
    @functools.partial(
        pl.kernel,
        out_type=jax.ShapeDtypeStruct((NPAD,), _f32),
        mesh=_sc_mesh(),
        compiler_params=_SC_PARAMS,
        scratch_types=[
            pltpu.VMEM((NE_TILE,), jnp.int32),
            pltpu.VMEM((NCH, CHUNK), jnp.int32),
            pltpu.VMEM((NPAD,), _f32),
            pltpu.VMEM((NE_TILE,), _f32),
            pltpu.VMEM((ROWS_T,), _f32),
            pltpu.VMEM((ROWS_T,), _f32),
            pltpu.VMEM((16,), _f32),
            pltpu.VMEM_SHARED((NPAD,), _f32),
            pltpu.SemaphoreType.DMA,
        ],
    )
    def k(v_hbm, di_hbm, b2_hbm, src_hbm, dst_hbm, out_hbm,
          src_v, dst_v, g_v, val_v, av, dv, b2v, acc_sh, ssem):
        sid = lax.axis_index("s")
        r0 = sid * ROWS_T
        pltpu.sync_copy(dst_hbm.at[sid], dst_v)
        pltpu.sync_copy(src_hbm.at[sid], src_v)
        pltpu.sync_copy(v_hbm.at[pl.ds(r0, ROWS_T)], acc_sh.at[pl.ds(r0, ROWS_T)])
        pltpu.sync_copy(v_hbm, g_v)
        pltpu.sync_copy(di_hbm.at[pl.ds(r0, ROWS_T)], dv)
        pltpu.sync_copy(b2_hbm, b2v)

        def gbody(i, carry):
            idx = src_v[pl.ds(i * 16, 16)]
            val_v[pl.ds(i * 16, 16)] = plsc.load_gather(g_v, [idx])
            return carry

        lax.fori_loop(0, NE_TILE // 16, gbody, 0, unroll=4)
        plsc.subcore_barrier()

        def group(g, carry):
            base = g * FIRE
            for t in range(FIRE):
                pltpu.make_async_copy(
                    val_v.at[pl.ds((base + t) * CHUNK, CHUNK)],
                    acc_sh.at[dst_v.at[base + t]], ssem).start(add=True)
            for t in range(FIRE):
                pltpu.make_async_copy(
                    val_v.at[pl.ds((base + t) * CHUNK, CHUNK)],
                    acc_sh.at[dst_v.at[base + t]], ssem).wait()
            return carry

        lax.fori_loop(0, NCH // FIRE, group, 0, unroll=False)
        plsc.subcore_barrier()

        pltpu.sync_copy(acc_sh.at[pl.ds(r0, ROWS_T)], av)
        b2vec = b2v[pl.ds(0, 16)]

        def fbody(i, carry):
            s = pl.ds(i * 16, 16)
            av[s] = av[s] * dv[s] + b2vec
            return carry

        lax.fori_loop(0, ROWS_T // 16, fbody, 0, unroll=4)
        pltpu.sync_copy(av, out_hbm.at[pl.ds(r0, ROWS_T)])

    return k(g2, dinv, b2b, srcf, dst3)


ROWS_B = 1280


def _tc_call(body, nout, *args):
    specs = []
    for a in args:
        if a.shape[0] == NPAD:
            specs.append(pl.BlockSpec((ROWS_B,) + a.shape[1:],
                                      lambda i: (i,) + (0,) * (a.ndim - 1)))
        else:
            specs.append(pl.BlockSpec(a.shape, lambda i: (0,) * a.ndim))
    out_shapes = [jax.ShapeDtypeStruct((NPAD, w), _f32) for w in nout]
    out_specs = [pl.BlockSpec((ROWS_B, w), lambda i: (i, 0)) for w in nout]
    return pl.pallas_call(
        body,
        grid=(NPAD // ROWS_B,),
        in_specs=specs,
        out_specs=out_specs,
        out_shape=out_shapes,
    )(*args)


def _tc_deg_g1(x, W1, deg):
    def body(x_ref, w_ref, d_ref, dinv_ref, g1_ref):
        dinv = lax.rsqrt(jnp.maximum(d_ref[...], 1.0))
        h0 = jnp.dot(x_ref[...], w_ref[...], preferred_element_type=_f32)
        dinv_ref[...] = dinv
        g1_ref[...] = dinv * h0

    return _tc_call(body, (1, HIDDEN), x, W1, deg)


def _tc_h_g2(ap, dinv, b1, W2):
    def body(a_ref, di_ref, b1_ref, w2_ref, g2_ref):
        dinv = di_ref[...]
        h = jnp.maximum(dinv * a_ref[...] + b1_ref[...], 0.0)
        z = jnp.dot(h, w2_ref[...], preferred_element_type=_f32)
        g2_ref[...] = dinv * z

    (g2,) = _tc_call(body, (1,), ap, dinv, b1, W2)
    return g2


_PAD_IDX = np.asarray(
    N_NODES + np.arange(EPAD - N_EDGES) % (NPAD - N_NODES), dtype=np.int32)


def kernel(x, edge_index, W1, b1, W2, b2):
    ei = edge_index.astype(jnp.int32)
    pad_idx = jnp.asarray(_PAD_IDX)
    src = jnp.concatenate([ei[0], pad_idx])
    dst = jnp.concatenate([ei[1], pad_idx])
    src3 = src.reshape(NS, NCH, CHUNK)
    dst3 = dst.reshape(NS, NCH, CHUNK)
    srcf = src.reshape(NS, NE_TILE)
    xp = jnp.pad(x, ((0, NPAD - N_NODES), (0, 0)))
    ones = jnp.ones((NPAD,), _f32)
    b1r = b1.reshape(1, HIDDEN)
    b2b = jnp.broadcast_to(b2, (16,))

    deg = _sc_deg(ones, dst3)
    dinv, g1 = _tc_deg_g1(xp, W1, deg.reshape(NPAD, 1))
    ap = _sc_scatter_rows(g1, src3, dst3)
    g2 = _tc_h_g2(ap, dinv, b1r, W2)
    out = _sc_scatter_scalar_final(g2.reshape(NPAD), dinv.reshape(NPAD),
                                   b2b, srcf, dst3)
    return out[:N_NODES]


# speedup vs baseline: 1.8334x; 1.2249x over previous
import functools

import jax
import jax.numpy as jnp
import numpy as np
from jax import lax
from jax.experimental import pallas as pl
from jax.experimental.pallas import tpu as pltpu
from jax.experimental.pallas import tpu_sc as plsc

N_NODES = 10000
D_FEAT = 128
HIDDEN = 32
N_EDGES = 320000

NPAD = 10240
NS = 16
CHUNK = 128
NCH = 157
NE_TILE = NCH * CHUNK
EPAD = NS * NE_TILE
ROWS_T = NPAD // NS
FIRE = 8
NBUF = 4

_f32 = jnp.float32


def _rsqrt16(x):
    i = plsc.bitcast(x, jnp.int32)
    i = 0x5F3759DF - lax.shift_right_logical(i, 1)
    y = plsc.bitcast(i, _f32)
    y = y * (1.5 - 0.5 * x * y * y)
    y = y * (1.5 - 0.5 * x * y * y)
    y = y * (1.5 - 0.5 * x * y * y)
    return y


def _sc_mega(h0, b1, w2, b2b, srcf, dst3):
    @functools.partial(
        pl.kernel,
        out_type=(jax.ShapeDtypeStruct((NPAD,), _f32),
                  jax.ShapeDtypeStruct((NPAD, HIDDEN), _f32)),
        mesh=plsc.VectorSubcoreMesh(core_axis_name="c", subcore_axis_name="s",
                                    num_cores=1),
        compiler_params=pltpu.CompilerParams(use_tc_tiling_on_sc=False,
                                             needs_layout_passes=False),
        scratch_types=[
            pltpu.VMEM((NE_TILE,), jnp.int32),
            pltpu.VMEM((NCH, CHUNK), jnp.int32),
            pltpu.VMEM((CHUNK,), _f32),
            pltpu.VMEM((NBUF, CHUNK, HIDDEN), _f32),
            pltpu.VMEM((ROWS_T, HIDDEN), _f32),
            pltpu.VMEM((ROWS_T,), _f32),
            pltpu.VMEM((ROWS_T,), _f32),
            pltpu.VMEM((NPAD,), _f32),
            pltpu.VMEM((NE_TILE,), _f32),
            pltpu.VMEM((HIDDEN,), _f32),
            pltpu.VMEM((HIDDEN,), _f32),
            pltpu.VMEM((16,), _f32),
            pltpu.VMEM_SHARED((NPAD,), _f32),
            pltpu.VMEM_SHARED((NPAD, HIDDEN), _f32),
            pltpu.VMEM_SHARED((NPAD,), _f32),
            pltpu.SemaphoreType.DMA,
            pltpu.SemaphoreType.DMA,
            pltpu.SemaphoreType.DMA,
        ],
    )
    def k(h0_hbm, b1_hbm, w2_hbm, b2_hbm, src_hbm, dst_hbm,
          out_hbm, g1_hbm,
          src_v, dst_v, ones_v, bufs, slab, dv, sv, g_v, val_v,
          b1v, w2v, b2v, dacc_sh, acc_sh, g2_sh,
          gsem, ssem, hsem):
        deg_sh = dacc_sh
        acc2_sh = dacc_sh
        sid = lax.axis_index("s")
        r0 = sid * ROWS_T

        pltpu.make_async_copy(h0_hbm.at[pl.ds(r0, ROWS_T)], slab, hsem).start()
        pltpu.sync_copy(src_hbm.at[sid], src_v)
        pltpu.sync_copy(dst_hbm.at[sid], dst_v)
        pltpu.sync_copy(b1_hbm, b1v)
        pltpu.sync_copy(w2_hbm, w2v)
        pltpu.sync_copy(b2_hbm, b2v)
        one16 = jnp.full((16,), 1.0, dtype=_f32)
        for i in range(CHUNK // 16):
            ones_v[pl.ds(i * 16, 16)] = one16

        def fill_ones(i, carry):
            sv[pl.ds(i * 16, 16)] = one16
            return carry

        lax.fori_loop(0, ROWS_T // 16, fill_ones, 0, unroll=4)
        pltpu.sync_copy(sv, deg_sh.at[pl.ds(r0, ROWS_T)])
        plsc.subcore_barrier()

        def deg_group(g, carry):
            base = g * FIRE
            for t in range(FIRE):
                pltpu.make_async_copy(
                    ones_v, deg_sh.at[dst_v.at[base + t]], ssem).start(add=True)
            for t in range(FIRE):
                pltpu.make_async_copy(
                    ones_v, deg_sh.at[dst_v.at[base + t]], ssem).wait()
            return carry

        lax.fori_loop(0, NCH // FIRE, deg_group, 0, unroll=False)
        for t in range(NCH - NCH % FIRE, NCH):
            pltpu.make_async_copy(
                ones_v, deg_sh.at[dst_v.at[t]], ssem).start(add=True)
        for t in range(NCH - NCH % FIRE, NCH):
            pltpu.make_async_copy(
                ones_v, deg_sh.at[dst_v.at[t]], ssem).wait()
        plsc.subcore_barrier()

        pltpu.sync_copy(deg_sh.at[pl.ds(r0, ROWS_T)], dv)

        def rsq(i, carry):
            s = pl.ds(i * 16, 16)
            dv[s] = _rsqrt16(dv[s])
            return carry

        lax.fori_loop(0, ROWS_T // 16, rsq, 0, unroll=4)

        pltpu.make_async_copy(h0_hbm.at[pl.ds(r0, ROWS_T)], slab, hsem).wait()

        a = pl.ds(0, 16)
        b = pl.ds(16, 16)

        def scale_rows(i, carry):
            dvec = dv[pl.ds(i * 16, 16)]
            for t in range(16):
                r = i * 16 + t
                s = jnp.full((16,), dvec[t], dtype=_f32)
                slab[r, a] = slab[r, a] * s
                slab[r, b] = slab[r, b] * s
            return carry

        lax.fori_loop(0, ROWS_T // 16, scale_rows, 0, unroll=False)
        pltpu.sync_copy(slab, g1_hbm.at[pl.ds(r0, ROWS_T)])
        pltpu.sync_copy(slab, acc_sh.at[pl.ds(r0, ROWS_T)])
        plsc.subcore_barrier()

        pltpu.make_async_copy(
            g1_hbm.at[src_v.at[pl.ds(0, CHUNK)]], bufs.at[0], gsem).start()
        pltpu.make_async_copy(
            g1_hbm.at[src_v.at[pl.ds(CHUNK, CHUNK)]], bufs.at[1], gsem).start()

        def edge_body(j, carry):
            slot = lax.rem(j, NBUF)

            @pl.when(j >= 2)
            def _():
                pltpu.make_async_copy(
                    bufs.at[lax.rem(j - 2, NBUF)],
                    acc_sh.at[dst_v.at[j - 2]], ssem).wait()

            @pl.when(j < NCH - 2)
            def _():
                pltpu.make_async_copy(
                    g1_hbm.at[src_v.at[pl.ds((j + 2) * CHUNK, CHUNK)]],
                    bufs.at[lax.rem(j + 2, NBUF)], gsem).start()

            pltpu.make_async_copy(
                g1_hbm.at[src_v.at[pl.ds(j * CHUNK, CHUNK)]],
                bufs.at[slot], gsem).wait()
            pltpu.make_async_copy(
                bufs.at[slot], acc_sh.at[dst_v.at[j]], ssem).start(add=True)
            return carry

        lax.fori_loop(0, NCH, edge_body, 0, unroll=False)
        for j in (NCH - 2, NCH - 1):
            pltpu.make_async_copy(
                bufs.at[j % NBUF], acc_sh.at[dst_v.at[j]], ssem).wait()
        plsc.subcore_barrier()

        pltpu.sync_copy(acc_sh.at[pl.ds(r0, ROWS_T)], slab)
        b1a = b1v[pl.ds(0, 16)]
        b1b = b1v[pl.ds(16, 16)]
        w2a = w2v[pl.ds(0, 16)]
        w2b = w2v[pl.ds(16, 16)]
        zero16 = jnp.zeros((16,), _f32)
        lane = lax.iota(jnp.int32, 16)

        def row_dot(i, carry):
            dvec = dv[pl.ds(i * 16, 16)]
            zvec = zero16
            for t in range(16):
                r = i * 16 + t
                s = jnp.full((16,), dvec[t], dtype=_f32)
                ha = jnp.maximum(slab[r, a] * s + b1a, zero16)
                hb = jnp.maximum(slab[r, b] * s + b1b, zero16)
                z = jnp.sum(ha * w2a + hb * w2b)
                zvec = jnp.where(lane == t, z, zvec)
            sv[pl.ds(i * 16, 16)] = zvec * dvec
            return carry

        lax.fori_loop(0, ROWS_T // 16, row_dot, 0, unroll=False)
        pltpu.sync_copy(sv, g2_sh.at[pl.ds(r0, ROWS_T)])
        pltpu.sync_copy(sv, acc2_sh.at[pl.ds(r0, ROWS_T)])
        plsc.subcore_barrier()

        pltpu.sync_copy(g2_sh, g_v)

        def gbody(i, carry):
            s = pl.ds(i * 16, 16)
            val_v[s] = plsc.load_gather(g_v, [src_v[s]])
            return carry

        lax.fori_loop(0, NE_TILE // 16, gbody, 0, unroll=4)

        def sc_group(g, carry):
            base = g * FIRE
            for t in range(FIRE):
                pltpu.make_async_copy(
                    val_v.at[pl.ds((base + t) * CHUNK, CHUNK)],
                    acc2_sh.at[dst_v.at[base + t]], ssem).start(add=True)
            for t in range(FIRE):
                pltpu.make_async_copy(
                    val_v.at[pl.ds((base + t) * CHUNK, CHUNK)],
                    acc2_sh.at[dst_v.at[base + t]], ssem).wait()
            return carry

        lax.fori_loop(0, NCH // FIRE, sc_group, 0, unroll=False)
        for t in range(NCH - NCH % FIRE, NCH):
            pltpu.make_async_copy(
                val_v.at[pl.ds(t * CHUNK, CHUNK)],
                acc2_sh.at[dst_v.at[t]], ssem).start(add=True)
        for t in range(NCH - NCH % FIRE, NCH):
            pltpu.make_async_copy(
                val_v.at[pl.ds(t * CHUNK, CHUNK)],
                acc2_sh.at[dst_v.at[t]], ssem).wait()
        plsc.subcore_barrier()

        pltpu.sync_copy(acc2_sh.at[pl.ds(r0, ROWS_T)], sv)
        b2vec = b2v[pl.ds(0, 16)]

        def fin(i, carry):
            s = pl.ds(i * 16, 16)
            sv[s] = sv[s] * dv[s] + b2vec
            return carry

        lax.fori_loop(0, ROWS_T // 16, fin, 0, unroll=4)
        pltpu.sync_copy(sv, out_hbm.at[pl.ds(r0, ROWS_T)])

    return k(h0, b1, w2, b2b, srcf, dst3)


ROWS_B = 1280


def _tc_h0(x, W1):
    def body(x_ref, w_ref, o_ref):
        o_ref[...] = jnp.dot(x_ref[...], w_ref[...], preferred_element_type=_f32)

    return pl.pallas_call(
        body,
        grid=(NPAD // ROWS_B,),
        in_specs=[pl.BlockSpec((ROWS_B, D_FEAT), lambda i: (i, 0)),
                  pl.BlockSpec((D_FEAT, HIDDEN), lambda i: (0, 0))],
        out_specs=pl.BlockSpec((ROWS_B, HIDDEN), lambda i: (i, 0)),
        out_shape=jax.ShapeDtypeStruct((NPAD, HIDDEN), _f32),
    )(x, W1)


_PAD_IDX = np.asarray(
    N_NODES + np.arange(EPAD - N_EDGES) % (NPAD - N_NODES), dtype=np.int32)


def kernel(x, edge_index, W1, b1, W2, b2):
    ei = edge_index.astype(jnp.int32)
    pad_idx = jnp.asarray(_PAD_IDX)
    src = jnp.concatenate([ei[0], pad_idx])
    dst = jnp.concatenate([ei[1], pad_idx])
    srcf = src.reshape(NS, NE_TILE)
    dst3 = dst.reshape(NS, NCH, CHUNK)
    xp = jnp.pad(x, ((0, NPAD - N_NODES), (0, 0)))
    b2b = jnp.broadcast_to(b2, (16,))
    w2 = W2.reshape(HIDDEN)

    h0 = _tc_h0(xp, W1)
    out, _ = _sc_mega(h0, b1, w2, b2b, srcf, dst3)
    return out[:N_NODES]


# speedup vs baseline: 1.8929x; 1.0324x over previous
import functools

import jax
import jax.numpy as jnp
import numpy as np
from jax import lax
from jax.experimental import pallas as pl
from jax.experimental.pallas import tpu as pltpu
from jax.experimental.pallas import tpu_sc as plsc

N_NODES = 10000
D_FEAT = 128
HIDDEN = 32
N_EDGES = 320000

NPAD = 10240
NS = 16
CHUNK = 128
NCH = 157
NE_TILE = NCH * CHUNK
EPAD = NS * NE_TILE
ROWS_T = NPAD // NS
NBUF = 4
NROW = NPAD // 16
NGRP = ROWS_T // 16

_f32 = jnp.float32


def _rsqrt16(x):
    i = plsc.bitcast(x, jnp.int32)
    i = 0x5F3759DF - lax.shift_right_logical(i, 1)
    y = plsc.bitcast(i, _f32)
    y = y * (1.5 - 0.5 * x * y * y)
    y = y * (1.5 - 0.5 * x * y * y)
    y = y * (1.5 - 0.5 * x * y * y)
    return y


def _sc_mega(h0, b1, w2, b2b, srcf, dst3):
    @functools.partial(
        pl.kernel,
        out_type=(jax.ShapeDtypeStruct((NPAD,), _f32),
                  jax.ShapeDtypeStruct((NPAD, HIDDEN), _f32)),
        mesh=plsc.VectorSubcoreMesh(core_axis_name="c", subcore_axis_name="s",
                                    num_cores=1),
        compiler_params=pltpu.CompilerParams(use_tc_tiling_on_sc=False,
                                             needs_layout_passes=False),
        scratch_types=[
            pltpu.VMEM((NE_TILE,), jnp.int32),
            pltpu.VMEM((NCH, CHUNK), jnp.int32),
            pltpu.VMEM((NBUF, CHUNK, HIDDEN), _f32),
            pltpu.VMEM((ROWS_T, HIDDEN), _f32),
            pltpu.VMEM((NGRP, 16), _f32),
            pltpu.VMEM((ROWS_T,), _f32),
            pltpu.VMEM((NPAD,), _f32),
            pltpu.VMEM((NROW, 16), _f32),
            pltpu.VMEM((NROW,), jnp.int32),
            pltpu.VMEM((HIDDEN,), _f32),
            pltpu.VMEM((HIDDEN,), _f32),
            pltpu.VMEM((16,), _f32),
            pltpu.VMEM_SHARED((NROW, 16), _f32),
            pltpu.VMEM_SHARED((NPAD, HIDDEN), _f32),
            pltpu.VMEM_SHARED((NPAD,), _f32),
            pltpu.SemaphoreType.DMA,
            pltpu.SemaphoreType.DMA,
            pltpu.SemaphoreType.DMA,
        ],
    )
    def k(h0_hbm, b1_hbm, w2_hbm, b2_hbm, src_hbm, dst_hbm,
          out_hbm, g1_hbm,
          src_v, dst_v, bufs, slab, dv2, sv, g_v, hist, iotab,
          b1v, w2v, b2v, dacc_sh, acc_sh, g2_sh,
          gsem, ssem, hsem):
        sid = lax.axis_index("s")
        r0 = sid * ROWS_T
        hrow0 = sid * NGRP
        one16 = jnp.full((16,), 1.0, dtype=_f32)
        zero16 = jnp.zeros((16,), _f32)
        iota16 = lax.iota(jnp.int32, 16)

        pltpu.make_async_copy(h0_hbm.at[pl.ds(r0, ROWS_T)], slab, hsem).start()
        pltpu.sync_copy(src_hbm.at[sid], src_v)
        pltpu.sync_copy(dst_hbm.at[sid], dst_v)
        pltpu.sync_copy(b1_hbm, b1v)
        pltpu.sync_copy(w2_hbm, w2v)
        pltpu.sync_copy(b2_hbm, b2v)

        def fill_iota(i, carry):
            iotab[pl.ds(i * 16, 16)] = iota16 + i * 16
            return carry

        lax.fori_loop(0, NROW // 16, fill_iota, 0, unroll=4)

        def zero_hist(r, carry):
            hist[r, pl.ds(0, 16)] = zero16
            return carry

        lax.fori_loop(0, NROW, zero_hist, 0, unroll=8)

        def zero_dv(i, carry):
            dv2[i, pl.ds(0, 16)] = zero16
            return carry

        lax.fori_loop(0, NGRP, zero_dv, 0, unroll=4)
        pltpu.sync_copy(dv2, dacc_sh.at[pl.ds(hrow0, NGRP)])
        plsc.subcore_barrier()

        def deg_chunk(j, carry):
            for t in range(CHUNK // 16):
                d16 = dst_v[j, pl.ds(t * 16, 16)]
                r16 = lax.shift_right_logical(d16, 4)
                c16 = jnp.bitwise_and(d16, 15)
                plsc.addupdate_scatter(hist, [r16, c16], one16)
            return carry

        lax.fori_loop(0, NCH, deg_chunk, 0, unroll=False)
        pltpu.sync_copy(hist, dacc_sh.at[iotab], add=True)
        plsc.subcore_barrier()

        pltpu.sync_copy(dacc_sh.at[pl.ds(hrow0, NGRP)], dv2)

        def rsq(i, carry):
            v = dv2[i, pl.ds(0, 16)]
            dv2[i, pl.ds(0, 16)] = _rsqrt16(v + one16)
            return carry

        lax.fori_loop(0, NGRP, rsq, 0, unroll=4)

        pltpu.make_async_copy(h0_hbm.at[pl.ds(r0, ROWS_T)], slab, hsem).wait()
        a = pl.ds(0, 16)
        b = pl.ds(16, 16)

        def scale_rows(i, carry):
            dvec = dv2[i, pl.ds(0, 16)]
            for t in range(16):
                r = i * 16 + t
                s = jnp.full((16,), dvec[t], dtype=_f32)
                slab[r, a] = slab[r, a] * s
                slab[r, b] = slab[r, b] * s
            return carry

        lax.fori_loop(0, NGRP, scale_rows, 0, unroll=False)
        pltpu.sync_copy(slab, g1_hbm.at[pl.ds(r0, ROWS_T)])
        pltpu.sync_copy(slab, acc_sh.at[pl.ds(r0, ROWS_T)])
        plsc.subcore_barrier()

        pltpu.make_async_copy(
            g1_hbm.at[src_v.at[pl.ds(0, CHUNK)]], bufs.at[0], gsem).start()
        pltpu.make_async_copy(
            g1_hbm.at[src_v.at[pl.ds(CHUNK, CHUNK)]], bufs.at[1], gsem).start()

        def edge_body(j, carry):
            slot = lax.rem(j, NBUF)

            @pl.when(j >= 2)
            def _():
                pltpu.make_async_copy(
                    bufs.at[lax.rem(j - 2, NBUF)],
                    acc_sh.at[dst_v.at[j - 2]], ssem).wait()

            @pl.when(j < NCH - 2)
            def _():
                pltpu.make_async_copy(
                    g1_hbm.at[src_v.at[pl.ds((j + 2) * CHUNK, CHUNK)]],
                    bufs.at[lax.rem(j + 2, NBUF)], gsem).start()

            pltpu.make_async_copy(
                g1_hbm.at[src_v.at[pl.ds(j * CHUNK, CHUNK)]],
                bufs.at[slot], gsem).wait()
            pltpu.make_async_copy(
                bufs.at[slot], acc_sh.at[dst_v.at[j]], ssem).start(add=True)
            return carry

        lax.fori_loop(0, NCH, edge_body, 0, unroll=False)
        for j in (NCH - 2, NCH - 1):
            pltpu.make_async_copy(
                bufs.at[j % NBUF], acc_sh.at[dst_v.at[j]], ssem).wait()
        plsc.subcore_barrier()

        pltpu.sync_copy(acc_sh.at[pl.ds(r0, ROWS_T)], slab)
        b1a = b1v[a]
        b1b = b1v[b]
        w2a = w2v[a]
        w2b = w2v[b]

        def row_dot(i, carry):
            dvec = dv2[i, pl.ds(0, 16)]
            zvec = zero16
            for t in range(16):
                r = i * 16 + t
                s = jnp.full((16,), dvec[t], dtype=_f32)
                ha = jnp.maximum(slab[r, a] * s + b1a, zero16)
                hb = jnp.maximum(slab[r, b] * s + b1b, zero16)
                z = jnp.sum(ha * w2a + hb * w2b)
                zvec = jnp.where(iota16 == t, z, zvec)
            sv[pl.ds(i * 16, 16)] = zvec * dvec
            return carry

        lax.fori_loop(0, NGRP, row_dot, 0, unroll=False)
        pltpu.sync_copy(sv, g2_sh.at[pl.ds(r0, ROWS_T)])
        lax.fori_loop(0, NROW, zero_hist, 0, unroll=8)
        pltpu.sync_copy(hist.at[pl.ds(0, NGRP)], dacc_sh.at[pl.ds(hrow0, NGRP)])
        plsc.subcore_barrier()

        pltpu.sync_copy(g2_sh, g_v)

        def sc_chunk(j, carry):
            for t in range(CHUNK // 16):
                s16 = src_v[pl.ds(j * CHUNK + t * 16, 16)]
                vals = plsc.load_gather(g_v, [s16])
                d16 = dst_v[j, pl.ds(t * 16, 16)]
                r16 = lax.shift_right_logical(d16, 4)
                c16 = jnp.bitwise_and(d16, 15)
                plsc.addupdate_scatter(hist, [r16, c16], vals)
            return carry

        lax.fori_loop(0, NCH, sc_chunk, 0, unroll=False)
        pltpu.sync_copy(hist, dacc_sh.at[iotab], add=True)
        plsc.subcore_barrier()

        pltpu.sync_copy(dacc_sh.at[pl.ds(hrow0, NGRP)], hist.at[pl.ds(0, NGRP)])
        b2vec = b2v[a]

        def fin(i, carry):
            s = pl.ds(i * 16, 16)
            accv = hist[i, pl.ds(0, 16)]
            dvec = dv2[i, pl.ds(0, 16)]
            sv[s] = (accv + sv[s]) * dvec + b2vec
            return carry

        lax.fori_loop(0, NGRP, fin, 0, unroll=4)
        pltpu.sync_copy(sv, out_hbm.at[pl.ds(r0, ROWS_T)])

    return k(h0, b1, w2, b2b, srcf, dst3)


ROWS_B = 1280


def _tc_h0(x, W1):
    def body(x_ref, w_ref, o_ref):
        o_ref[...] = jnp.dot(x_ref[...], w_ref[...], preferred_element_type=_f32)

    return pl.pallas_call(
        body,
        grid=(NPAD // ROWS_B,),
        in_specs=[pl.BlockSpec((ROWS_B, D_FEAT), lambda i: (i, 0)),
                  pl.BlockSpec((D_FEAT, HIDDEN), lambda i: (0, 0))],
        out_specs=pl.BlockSpec((ROWS_B, HIDDEN), lambda i: (i, 0)),
        out_shape=jax.ShapeDtypeStruct((NPAD, HIDDEN), _f32),
    )(x, W1)


_PAD_IDX = np.asarray(
    N_NODES + np.arange(EPAD - N_EDGES) % (NPAD - N_NODES), dtype=np.int32)


def kernel(x, edge_index, W1, b1, W2, b2):
    ei = edge_index.astype(jnp.int32)
    pad_idx = jnp.asarray(_PAD_IDX)
    src = jnp.concatenate([ei[0], pad_idx])
    dst = jnp.concatenate([ei[1], pad_idx])
    srcf = src.reshape(NS, NE_TILE)
    dst3 = dst.reshape(NS, NCH, CHUNK)
    xp = jnp.pad(x, ((0, NPAD - N_NODES), (0, 0)))
    b2b = jnp.broadcast_to(b2, (16,))
    w2 = W2.reshape(HIDDEN)

    h0 = _tc_h0(xp, W1)
    out, _ = _sc_mega(h0, b1, w2, b2b, srcf, dst3)
    return out[:N_NODES]


# speedup vs baseline: 1.9822x; 1.0472x over previous
import functools

import jax
import jax.numpy as jnp
import numpy as np
from jax import lax
from jax.experimental import pallas as pl
from jax.experimental.pallas import tpu as pltpu
from jax.experimental.pallas import tpu_sc as plsc

N_NODES = 10000
D_FEAT = 128
HIDDEN = 32
N_EDGES = 320000

NPAD = 10240
NS = 16
CHUNK = 128
NCH = 157
NE_TILE = NCH * CHUNK
EPAD = NS * NE_TILE
ROWS_T = NPAD // NS
NBUF = 6
AHEAD = 3
NROW = NPAD // 16
NGRP = ROWS_T // 16

_f32 = jnp.float32


def _rsqrt16(x):
    i = plsc.bitcast(x, jnp.int32)
    i = 0x5F3759DF - lax.shift_right_logical(i, 1)
    y = plsc.bitcast(i, _f32)
    y = y * (1.5 - 0.5 * x * y * y)
    y = y * (1.5 - 0.5 * x * y * y)
    y = y * (1.5 - 0.5 * x * y * y)
    return y


def _sc_mega(h0, b1, w2, b2b, srcf, dst3):
    @functools.partial(
        pl.kernel,
        out_type=(jax.ShapeDtypeStruct((NPAD,), _f32),
                  jax.ShapeDtypeStruct((NPAD, HIDDEN), _f32)),
        mesh=plsc.VectorSubcoreMesh(core_axis_name="c", subcore_axis_name="s",
                                    num_cores=1),
        compiler_params=pltpu.CompilerParams(use_tc_tiling_on_sc=False,
                                             needs_layout_passes=False),
        scratch_types=[
            pltpu.VMEM((NE_TILE,), jnp.int32),
            pltpu.VMEM((NCH, CHUNK), jnp.int32),
            pltpu.VMEM((NBUF, CHUNK, HIDDEN), _f32),
            pltpu.VMEM((ROWS_T, HIDDEN), _f32),
            pltpu.VMEM((NGRP, 16), _f32),
            pltpu.VMEM((ROWS_T,), _f32),
            pltpu.VMEM((NPAD,), _f32),
            pltpu.VMEM((NROW, 16), _f32),
            pltpu.VMEM((NROW,), jnp.int32),
            pltpu.VMEM((HIDDEN,), _f32),
            pltpu.VMEM((HIDDEN,), _f32),
            pltpu.VMEM((16,), _f32),
            pltpu.VMEM_SHARED((NROW, 16), _f32),
            pltpu.VMEM_SHARED((NPAD, HIDDEN), _f32),
            pltpu.VMEM_SHARED((NPAD,), _f32),
            pltpu.SemaphoreType.DMA,
            pltpu.SemaphoreType.DMA,
            pltpu.SemaphoreType.DMA,
        ],
    )
    def k(h0_hbm, b1_hbm, w2_hbm, b2_hbm, src_hbm, dst_hbm,
          out_hbm, g1_hbm,
          src_v, dst_v, bufs, slab, dv2, sv, g_v, hist, iotab,
          b1v, w2v, b2v, dacc_sh, acc_sh, g2_sh,
          gsem, ssem, hsem):
        sid = lax.axis_index("s")
        r0 = sid * ROWS_T
        hrow0 = sid * NGRP
        one16 = jnp.full((16,), 1.0, dtype=_f32)
        zero16 = jnp.zeros((16,), _f32)
        iota16 = lax.iota(jnp.int32, 16)

        pltpu.make_async_copy(h0_hbm.at[pl.ds(r0, ROWS_T)], slab, hsem).start()
        pltpu.sync_copy(src_hbm.at[sid], src_v)
        pltpu.sync_copy(dst_hbm.at[sid], dst_v)
        pltpu.sync_copy(b1_hbm, b1v)
        pltpu.sync_copy(w2_hbm, w2v)
        pltpu.sync_copy(b2_hbm, b2v)

        def fill_iota(i, carry):
            iotab[pl.ds(i * 16, 16)] = iota16 + i * 16
            return carry

        lax.fori_loop(0, NROW // 16, fill_iota, 0, unroll=4)

        def zero_hist(r, carry):
            hist[r, pl.ds(0, 16)] = zero16
            return carry

        lax.fori_loop(0, NROW, zero_hist, 0, unroll=8)

        def zero_dv(i, carry):
            dv2[i, pl.ds(0, 16)] = zero16
            return carry

        lax.fori_loop(0, NGRP, zero_dv, 0, unroll=4)
        pltpu.sync_copy(dv2, dacc_sh.at[pl.ds(hrow0, NGRP)])
        plsc.subcore_barrier()

        def deg_chunk(j, carry):
            for t in range(CHUNK // 16):
                d16 = dst_v[j, pl.ds(t * 16, 16)]
                r16 = lax.shift_right_logical(d16, 4)
                c16 = jnp.bitwise_and(d16, 15)
                plsc.addupdate_scatter(hist, [r16, c16], one16)
            return carry

        lax.fori_loop(0, NCH, deg_chunk, 0, unroll=False)
        pltpu.sync_copy(hist, dacc_sh.at[iotab], add=True)
        plsc.subcore_barrier()

        pltpu.sync_copy(dacc_sh.at[pl.ds(hrow0, NGRP)], dv2)

        def rsq(i, carry):
            v = dv2[i, pl.ds(0, 16)]
            dv2[i, pl.ds(0, 16)] = _rsqrt16(v + one16)
            return carry

        lax.fori_loop(0, NGRP, rsq, 0, unroll=4)

        pltpu.make_async_copy(h0_hbm.at[pl.ds(r0, ROWS_T)], slab, hsem).wait()
        a = pl.ds(0, 16)
        b = pl.ds(16, 16)

        def scale_rows(i, carry):
            dvec = dv2[i, pl.ds(0, 16)]
            for t in range(16):
                r = i * 16 + t
                s = jnp.full((16,), dvec[t], dtype=_f32)
                slab[r, a] = slab[r, a] * s
                slab[r, b] = slab[r, b] * s
            return carry

        lax.fori_loop(0, NGRP, scale_rows, 0, unroll=False)
        pltpu.sync_copy(slab, g1_hbm.at[pl.ds(r0, ROWS_T)])
        pltpu.sync_copy(slab, acc_sh.at[pl.ds(r0, ROWS_T)])
        plsc.subcore_barrier()

        for p in range(AHEAD):
            pltpu.make_async_copy(
                g1_hbm.at[src_v.at[pl.ds(p * CHUNK, CHUNK)]],
                bufs.at[p], gsem).start()

        def edge_body(j, carry):
            slot = lax.rem(j, NBUF)

            @pl.when(j >= AHEAD)
            def _():
                pltpu.make_async_copy(
                    bufs.at[lax.rem(j - AHEAD, NBUF)],
                    acc_sh.at[dst_v.at[j - AHEAD]], ssem).wait()

            @pl.when(j < NCH - AHEAD)
            def _():
                pltpu.make_async_copy(
                    g1_hbm.at[src_v.at[pl.ds((j + AHEAD) * CHUNK, CHUNK)]],
                    bufs.at[lax.rem(j + AHEAD, NBUF)], gsem).start()

            pltpu.make_async_copy(
                g1_hbm.at[src_v.at[pl.ds(j * CHUNK, CHUNK)]],
                bufs.at[slot], gsem).wait()
            pltpu.make_async_copy(
                bufs.at[slot], acc_sh.at[dst_v.at[j]], ssem).start(add=True)
            return carry

        lax.fori_loop(0, NCH, edge_body, 0, unroll=2)
        for j in range(NCH - AHEAD, NCH):
            pltpu.make_async_copy(
                bufs.at[j % NBUF], acc_sh.at[dst_v.at[j]], ssem).wait()
        plsc.subcore_barrier()

        pltpu.sync_copy(acc_sh.at[pl.ds(r0, ROWS_T)], slab)
        b1a = b1v[a]
        b1b = b1v[b]
        w2a = w2v[a]
        w2b = w2v[b]

        def row_dot(i, carry):
            dvec = dv2[i, pl.ds(0, 16)]
            zvec = zero16
            for t in range(16):
                r = i * 16 + t
                s = jnp.full((16,), dvec[t], dtype=_f32)
                ha = jnp.maximum(slab[r, a] * s + b1a, zero16)
                hb = jnp.maximum(slab[r, b] * s + b1b, zero16)
                z = jnp.sum(ha * w2a + hb * w2b)
                zvec = jnp.where(iota16 == t, z, zvec)
            sv[pl.ds(i * 16, 16)] = zvec * dvec
            return carry

        lax.fori_loop(0, NGRP, row_dot, 0, unroll=False)
        pltpu.sync_copy(sv, g2_sh.at[pl.ds(r0, ROWS_T)])
        lax.fori_loop(0, NROW, zero_hist, 0, unroll=8)
        pltpu.sync_copy(hist.at[pl.ds(0, NGRP)], dacc_sh.at[pl.ds(hrow0, NGRP)])
        plsc.subcore_barrier()

        pltpu.sync_copy(g2_sh, g_v)

        def sc_chunk(j, carry):
            for t in range(CHUNK // 16):
                s16 = src_v[pl.ds(j * CHUNK + t * 16, 16)]
                vals = plsc.load_gather(g_v, [s16])
                d16 = dst_v[j, pl.ds(t * 16, 16)]
                r16 = lax.shift_right_logical(d16, 4)
                c16 = jnp.bitwise_and(d16, 15)
                plsc.addupdate_scatter(hist, [r16, c16], vals)
            return carry

        lax.fori_loop(0, NCH, sc_chunk, 0, unroll=False)
        pltpu.sync_copy(hist, dacc_sh.at[iotab], add=True)
        plsc.subcore_barrier()

        pltpu.sync_copy(dacc_sh.at[pl.ds(hrow0, NGRP)], hist.at[pl.ds(0, NGRP)])
        b2vec = b2v[a]

        def fin(i, carry):
            s = pl.ds(i * 16, 16)
            accv = hist[i, pl.ds(0, 16)]
            dvec = dv2[i, pl.ds(0, 16)]
            sv[s] = (accv + sv[s]) * dvec + b2vec
            return carry

        lax.fori_loop(0, NGRP, fin, 0, unroll=4)
        pltpu.sync_copy(sv, out_hbm.at[pl.ds(r0, ROWS_T)])

    return k(h0, b1, w2, b2b, srcf, dst3)


ROWS_B = 1280


def _tc_h0(x, W1):
    def body(x_ref, w_ref, o_ref):
        o_ref[...] = jnp.dot(x_ref[...], w_ref[...], preferred_element_type=_f32)

    return pl.pallas_call(
        body,
        grid=(NPAD // ROWS_B,),
        in_specs=[pl.BlockSpec((ROWS_B, D_FEAT), lambda i: (i, 0)),
                  pl.BlockSpec((D_FEAT, HIDDEN), lambda i: (0, 0))],
        out_specs=pl.BlockSpec((ROWS_B, HIDDEN), lambda i: (i, 0)),
        out_shape=jax.ShapeDtypeStruct((NPAD, HIDDEN), _f32),
    )(x, W1)


_PAD_IDX = np.asarray(
    N_NODES + np.arange(EPAD - N_EDGES) % (NPAD - N_NODES), dtype=np.int32)


def kernel(x, edge_index, W1, b1, W2, b2):
    ei = edge_index.astype(jnp.int32)
    pad_idx = jnp.asarray(_PAD_IDX)
    src = jnp.concatenate([ei[0], pad_idx])
    dst = jnp.concatenate([ei[1], pad_idx])
    srcf = src.reshape(NS, NE_TILE)
    dst3 = dst.reshape(NS, NCH, CHUNK)
    xp = jnp.pad(x, ((0, NPAD - N_NODES), (0, 0)))
    b2b = jnp.broadcast_to(b2, (16,))
    w2 = W2.reshape(HIDDEN)

    h0 = _tc_h0(xp, W1)
    out, _ = _sc_mega(h0, b1, w2, b2b, srcf, dst3)
    return out[:N_NODES]
